# Initial kernel scaffold; baseline (speedup 1.0000x reference)
import jax
import jax.numpy as jnp
from jax.experimental import pallas as pl


def kernel(token_ids, weights):
    raise NotImplementedError("write your pallas kernel here")


# speedup vs baseline: 4.4751x; 4.4751x over previous
import functools

import jax
import jax.numpy as jnp
from jax import lax
from jax.experimental import pallas as pl
from jax.experimental.pallas import tpu as pltpu
from jax.experimental.pallas import tpu_sc as plsc

B, L, T, V = 1024, 26, 50, 1001
ROWS = B * L
TP = 64
RPG = 8
GROUPS = ROWS // RPG
CHUNK = RPG * V
BUF = 8016
NC, NS = 2, 16
WORKERS = NC * NS
GPW = GROUPS // WORKERS
VPR = TP // 16
MAGIC = 0x5F3759DF


def _sc_body(tok_hbm, w_hbm, out_hbm, tok_v, w_v, buf):
    wid = lax.axis_index("s") * NC + lax.axis_index("c")
    base = wid * GPW

    def zero_body(i, carry):
        buf[pl.ds(pl.multiple_of(i * 16, 16), 16)] = jnp.zeros((16,), jnp.float32)
        return carry

    lax.fori_loop(0, BUF // 16, zero_body, 0)

    def group_body(i, carry):
        g = base + i
        pltpu.sync_copy(tok_hbm.at[g], tok_v)
        pltpu.sync_copy(w_hbm.at[g], w_v)

        idx = []
        for k in range(RPG * VPR):
            tok = tok_v[pl.ds(k * 16, 16)]
            ix = tok + jnp.int32((k // VPR) * V)
            idx.append(ix)
            plsc.addupdate_scatter(buf, [ix], w_v[pl.ds(k * 16, 16)])

        for j in range(RPG):
            cs = [plsc.load_gather(buf, [idx[j * VPR + m]]) for m in range(VPR)]
            part = jnp.zeros((16,), jnp.float32)
            for m in range(VPR):
                part = part + cs[m] * w_v[pl.ds((j * VPR + m) * 16, 16)]
            s = jnp.maximum(jnp.sum(part), 1e-12)
            xv = jnp.broadcast_to(s, (16,))
            iv = jnp.int32(MAGIC) - lax.shift_right_logical(
                plsc.bitcast(xv, jnp.int32), 1)
            y = plsc.bitcast(iv, jnp.float32)
            for _ in range(3):
                y = y * (1.5 - 0.5 * xv * y * y)
            for m in range(VPR):
                plsc.store_scatter(buf, [idx[j * VPR + m]], cs[m] * y)

        pltpu.sync_copy(buf.at[pl.ds(0, CHUNK)],
                        out_hbm.at[pl.ds(g * CHUNK, CHUNK)])
        zv = jnp.zeros((16,), jnp.float32)
        for k in range(RPG * VPR):
            plsc.store_scatter(buf, [idx[k]], zv)
        return carry

    lax.fori_loop(0, GPW, group_body, 0)


@jax.jit
def kernel(token_ids, weights):
    tok2 = token_ids.reshape(ROWS, T)
    w2 = weights.reshape(ROWS, T)
    tok_pad = jnp.concatenate(
        [tok2, jnp.broadcast_to(tok2[:, :1], (ROWS, TP - T))], axis=1)
    w_pad = jnp.concatenate(
        [w2, jnp.zeros((ROWS, TP - T), jnp.float32)], axis=1)
    tok_g = tok_pad.reshape(GROUPS, RPG * TP)
    w_g = w_pad.reshape(GROUPS, RPG * TP)

    mesh = plsc.VectorSubcoreMesh(
        core_axis_name="c", subcore_axis_name="s", num_cores=NC,
        num_subcores=NS)
    out = pl.kernel(
        _sc_body,
        out_type=jax.ShapeDtypeStruct((GROUPS * CHUNK,), jnp.float32),
        mesh=mesh,
        compiler_params=pltpu.CompilerParams(needs_layout_passes=False),
        scratch_types=[
            pltpu.VMEM((RPG * TP,), jnp.int32),
            pltpu.VMEM((RPG * TP,), jnp.float32),
            pltpu.VMEM((BUF,), jnp.float32),
        ],
    )(tok_g, w_g)
    return out.reshape(B, L, V)


# speedup vs baseline: 5.3208x; 1.1890x over previous
import jax
import jax.numpy as jnp
from jax import lax
from jax.experimental import pallas as pl
from jax.experimental.pallas import tpu as pltpu
from jax.experimental.pallas import tpu_sc as plsc

B, L, T, V = 1024, 26, 50, 1001
ROWS = B * L
TP = 64
RPG = 8
GROUPS = ROWS // RPG
CHUNK = RPG * V
BUF = 8016
GTOK = RPG * TP
NC, NS = 2, 16
WORKERS = NC * NS
GPW = GROUPS // WORKERS
PAIRS = GPW // 2
NVEC = GTOK // 16
VPR = TP // 16
MAGIC = 0x5F3759DF


def _fill_group(tok_v, w_v, buf, idx_v):
    idx = []
    for k in range(NVEC):
        tok = tok_v[pl.ds(k * 16, 16)]
        ix = tok + jnp.int32((k // VPR) * V)
        idx.append(ix)
        idx_v[pl.ds(k * 16, 16)] = ix
        plsc.addupdate_scatter(buf, [ix], w_v[pl.ds(k * 16, 16)])
    for j in range(RPG):
        cs = [plsc.load_gather(buf, [idx[j * VPR + m]]) for m in range(VPR)]
        part = jnp.zeros((16,), jnp.float32)
        for m in range(VPR):
            part = part + cs[m] * w_v[pl.ds((j * VPR + m) * 16, 16)]
        s = jnp.maximum(jnp.sum(part), 1e-12)
        xv = jnp.broadcast_to(s, (16,))
        iv = jnp.int32(MAGIC) - lax.shift_right_logical(
            plsc.bitcast(xv, jnp.int32), 1)
        y = plsc.bitcast(iv, jnp.float32)
        for _ in range(3):
            y = y * (1.5 - 0.5 * xv * y * y)
        for m in range(VPR):
            plsc.store_scatter(buf, [idx[j * VPR + m]], cs[m] * y)


def _rezero(buf, idx_v):
    zv = jnp.zeros((16,), jnp.float32)
    for k in range(NVEC):
        plsc.store_scatter(buf, [idx_v[pl.ds(k * 16, 16)]], zv)


def _sc_body(tok_hbm, w_hbm, out_hbm,
             tok0, tok1, w0, w1, buf0, buf1, idx0, idx1,
             osem0, osem1, tsem0, tsem1, wsem0, wsem1):
    wid = lax.axis_index("s") * NC + lax.axis_index("c")
    base = wid * GPW
    bufs = ((tok0, w0, buf0, idx0, osem0, tsem0, wsem0),
            (tok1, w1, buf1, idx1, osem1, tsem1, wsem1))

    def zero_body(i, carry):
        off = pl.multiple_of(i * 16, 16)
        buf0[pl.ds(off, 16)] = jnp.zeros((16,), jnp.float32)
        buf1[pl.ds(off, 16)] = jnp.zeros((16,), jnp.float32)
        return carry

    lax.fori_loop(0, BUF // 16, zero_body, 0)

    for b in range(2):
        tok_v, w_v, _, _, _, tsem, wsem = bufs[b]
        g = base + b
        pltpu.async_copy(tok_hbm.at[g], tok_v, tsem)
        pltpu.async_copy(w_hbm.at[g], w_v, wsem)

    def pair_body(i2, carry):
        for b in range(2):
            tok_v, w_v, buf, idx_v, osem, tsem, wsem = bufs[b]
            g = base + i2 * 2 + b

            @pl.when(i2 > 0)
            def _reclaim():
                pltpu.make_async_copy(
                    buf.at[pl.ds(0, CHUNK)],
                    out_hbm.at[pl.ds((g - 2) * CHUNK, CHUNK)], osem).wait()
                _rezero(buf, idx_v)

            pltpu.make_async_copy(tok_hbm.at[g], tok_v, tsem).wait()
            pltpu.make_async_copy(w_hbm.at[g], w_v, wsem).wait()

            _fill_group(tok_v, w_v, buf, idx_v)

            @pl.when(i2 < PAIRS - 1)
            def _prefetch():
                pltpu.async_copy(tok_hbm.at[g + 2], tok_v, tsem)
                pltpu.async_copy(w_hbm.at[g + 2], w_v, wsem)

            pltpu.async_copy(buf.at[pl.ds(0, CHUNK)],
                             out_hbm.at[pl.ds(g * CHUNK, CHUNK)], osem)
        return carry

    lax.fori_loop(0, PAIRS, pair_body, 0)

    for b in range(2):
        _, _, buf, _, osem, _, _ = bufs[b]
        g = base + (PAIRS - 1) * 2 + b
        pltpu.make_async_copy(buf.at[pl.ds(0, CHUNK)],
                              out_hbm.at[pl.ds(g * CHUNK, CHUNK)], osem).wait()


@jax.jit
def kernel(token_ids, weights):
    tok2 = token_ids.reshape(ROWS, T)
    w2 = weights.reshape(ROWS, T)
    tok_pad = jnp.concatenate(
        [tok2, jnp.broadcast_to(tok2[:, :1], (ROWS, TP - T))], axis=1)
    w_pad = jnp.concatenate(
        [w2, jnp.zeros((ROWS, TP - T), jnp.float32)], axis=1)
    tok_g = tok_pad.reshape(GROUPS, GTOK)
    w_g = w_pad.reshape(GROUPS, GTOK)

    mesh = plsc.VectorSubcoreMesh(
        core_axis_name="c", subcore_axis_name="s", num_cores=NC,
        num_subcores=NS)
    out = pl.kernel(
        _sc_body,
        out_type=jax.ShapeDtypeStruct((GROUPS * CHUNK,), jnp.float32),
        mesh=mesh,
        compiler_params=pltpu.CompilerParams(needs_layout_passes=False),
        scratch_types=[
            pltpu.VMEM((GTOK,), jnp.int32),
            pltpu.VMEM((GTOK,), jnp.int32),
            pltpu.VMEM((GTOK,), jnp.float32),
            pltpu.VMEM((GTOK,), jnp.float32),
            pltpu.VMEM((BUF,), jnp.float32),
            pltpu.VMEM((BUF,), jnp.float32),
            pltpu.VMEM((GTOK,), jnp.int32),
            pltpu.VMEM((GTOK,), jnp.int32),
            pltpu.SemaphoreType.DMA,
            pltpu.SemaphoreType.DMA,
            pltpu.SemaphoreType.DMA,
            pltpu.SemaphoreType.DMA,
            pltpu.SemaphoreType.DMA,
            pltpu.SemaphoreType.DMA,
        ],
    )(tok_g, w_g)
    return out.reshape(B, L, V)


# speedup vs baseline: 8.9251x; 1.6774x over previous
import jax
import jax.numpy as jnp
from jax import lax
from jax.experimental import pallas as pl
from jax.experimental.pallas import tpu as pltpu
from jax.experimental.pallas import tpu_sc as plsc

B, L, T, V = 1024, 26, 50, 1001
TP = 64
ROWS = B * L
LT_FULL = L // 8
LTAIL = L - 8 * LT_FULL
BTOK = L * TP
NC, NS = 2, 16
WORKERS = NC * NS
BPW = B // WORKERS
VPR = TP // 16
MAGIC = 0x5F3759DF


def _process_unit(tok_v, w_v, buf, lt, nrows):
    idx = []
    for r in range(nrows):
        for m in range(VPR):
            o = (8 * lt + r) * TP + m * 16
            tok = tok_v[pl.ds(o, 16)]
            rv = jnp.full((16,), r, jnp.int32)
            idx.append((rv, tok))
            plsc.addupdate_scatter(buf, [rv, tok], w_v[pl.ds(o, 16)])
    for r in range(nrows):
        cs = [plsc.load_gather(buf, list(idx[r * VPR + m])) for m in range(VPR)]
        part = jnp.zeros((16,), jnp.float32)
        for m in range(VPR):
            o = (8 * lt + r) * TP + m * 16
            part = part + cs[m] * w_v[pl.ds(o, 16)]
        s = jnp.maximum(jnp.sum(part), 1e-12)
        xv = jnp.broadcast_to(s, (16,))
        iv = jnp.int32(MAGIC) - lax.shift_right_logical(
            plsc.bitcast(xv, jnp.int32), 1)
        y = plsc.bitcast(iv, jnp.float32)
        for _ in range(3):
            y = y * (1.5 - 0.5 * xv * y * y)
        for m in range(VPR):
            plsc.store_scatter(buf, list(idx[r * VPR + m]), cs[m] * y)
    return idx


def _rezero(buf, idx):
    zv = jnp.zeros((16,), jnp.float32)
    for rv, tok in idx:
        plsc.store_scatter(buf, [rv, tok], zv)


def _sc_body(tok_hbm, w_hbm, out_hbm, tok_v, w_v, buf):
    wid = lax.axis_index("s") * NC + lax.axis_index("c")
    b0 = wid * BPW

    lanes = lax.iota(jnp.int32, 16)

    def zero_body(i, carry):
        r = jnp.broadcast_to(i // 63, (16,))
        c = (i % 63) * 16 + lanes
        plsc.store_scatter(buf, [r, c], jnp.zeros((16,), jnp.float32),
                           mask=c < V)
        return carry

    lax.fori_loop(0, 8 * 63, zero_body, 0)

    def b_body(i, carry):
        b = b0 + i
        pltpu.sync_copy(tok_hbm.at[pl.ds(b * BTOK, BTOK)], tok_v)
        pltpu.sync_copy(w_hbm.at[pl.ds(b * BTOK, BTOK)], w_v)
        for lt in range(LT_FULL):
            idx = _process_unit(tok_v, w_v, buf, lt, 8)
            pltpu.sync_copy(buf, out_hbm.at[b, pl.ds(8 * lt, 8), :])
            _rezero(buf, idx)
        idx = _process_unit(tok_v, w_v, buf, LT_FULL, LTAIL)
        pltpu.sync_copy(buf.at[pl.ds(0, LTAIL)],
                        out_hbm.at[b, pl.ds(8 * LT_FULL, LTAIL), :])
        _rezero(buf, idx)
        return carry

    lax.fori_loop(0, BPW, b_body, 0)


@jax.jit
def kernel(token_ids, weights):
    tok2 = token_ids.reshape(ROWS, T)
    w2 = weights.reshape(ROWS, T)
    tok_pad = jnp.concatenate(
        [tok2, jnp.broadcast_to(tok2[:, :1], (ROWS, TP - T))], axis=1)
    w_pad = jnp.concatenate(
        [w2, jnp.zeros((ROWS, TP - T), jnp.float32)], axis=1)
    tok_g = tok_pad.reshape(ROWS * TP)
    w_g = w_pad.reshape(ROWS * TP)

    mesh = plsc.VectorSubcoreMesh(
        core_axis_name="c", subcore_axis_name="s", num_cores=NC,
        num_subcores=NS)
    return pl.kernel(
        _sc_body,
        out_type=jax.ShapeDtypeStruct((B, L, V), jnp.float32),
        mesh=mesh,
        compiler_params=pltpu.CompilerParams(needs_layout_passes=False),
        scratch_types=[
            pltpu.VMEM((BTOK,), jnp.int32),
            pltpu.VMEM((BTOK,), jnp.float32),
            pltpu.VMEM((8, V), jnp.float32),
        ],
    )(tok_g, w_g)


# speedup vs baseline: 9.0073x; 1.0092x over previous
import jax
import jax.numpy as jnp
from jax import lax
from jax.experimental import pallas as pl
from jax.experimental.pallas import tpu as pltpu
from jax.experimental.pallas import tpu_sc as plsc

B, L, T, V = 1024, 26, 50, 1001
TP = 64
ROWS = B * L
LT_FULL = L // 8
LTAIL = L - 8 * LT_FULL
BTOK = L * TP
NC, NS = 2, 16
WORKERS = NC * NS
BPW = B // WORKERS
JP = BPW // 2
VPR = TP // 16
MAGIC = 0x5F3759DF

UNIT_ROWS = (8, 8, 8, LTAIL)
UNIT_L0 = (0, 8, 16, 24)


def _process_unit(tok_v, w_v, buf, tsave, lt):
    nrows = UNIT_ROWS[lt]
    idx = []
    for r in range(nrows):
        for m in range(VPR):
            o = (UNIT_L0[lt] + r) * TP + m * 16
            tok = tok_v[pl.ds(o, 16)]
            rv = jnp.full((16,), r, jnp.int32)
            idx.append((rv, tok))
            tsave[pl.ds((r * VPR + m) * 16, 16)] = tok
            plsc.addupdate_scatter(buf, [rv, tok], w_v[pl.ds(o, 16)])
    for r in range(nrows):
        cs = [plsc.load_gather(buf, list(idx[r * VPR + m])) for m in range(VPR)]
        part = jnp.zeros((16,), jnp.float32)
        for m in range(VPR):
            o = (UNIT_L0[lt] + r) * TP + m * 16
            part = part + cs[m] * w_v[pl.ds(o, 16)]
        s = jnp.maximum(jnp.sum(part), 1e-12)
        xv = jnp.broadcast_to(s, (16,))
        iv = jnp.int32(MAGIC) - lax.shift_right_logical(
            plsc.bitcast(xv, jnp.int32), 1)
        y = plsc.bitcast(iv, jnp.float32)
        for _ in range(3):
            y = y * (1.5 - 0.5 * xv * y * y)
        for m in range(VPR):
            plsc.store_scatter(buf, list(idx[r * VPR + m]), cs[m] * y)


def _rezero(buf, tsave, nrows):
    zv = jnp.zeros((16,), jnp.float32)
    for r in range(nrows):
        for m in range(VPR):
            rv = jnp.full((16,), r, jnp.int32)
            tok = tsave[pl.ds((r * VPR + m) * 16, 16)]
            plsc.store_scatter(buf, [rv, tok], zv)


def _out_slice(out_hbm, b, lt):
    return out_hbm.at[b, pl.ds(UNIT_L0[lt], UNIT_ROWS[lt]), :]


def _buf_slice(buf, lt):
    return buf.at[pl.ds(0, UNIT_ROWS[lt])] if UNIT_ROWS[lt] != 8 else buf


def _sc_body(tok_hbm, w_hbm, out_hbm,
             tok0, tok1, w0, w1, buf0, buf1, tsv0, tsv1,
             osem0, osem1, tsem0, tsem1, wsem0, wsem1):
    wid = lax.axis_index("s") * NC + lax.axis_index("c")
    b0 = wid * BPW
    bufs = (buf0, buf1)
    tsvs = (tsv0, tsv1)
    osems = (osem0, osem1)
    ins = ((tok0, w0, tsem0, wsem0), (tok1, w1, tsem1, wsem1))

    lanes = lax.iota(jnp.int32, 16)

    def zero_body(i, carry):
        r = jnp.broadcast_to(i // 63, (16,))
        c = (i % 63) * 16 + lanes
        zv = jnp.zeros((16,), jnp.float32)
        plsc.store_scatter(buf0, [r, c], zv, mask=c < V)
        plsc.store_scatter(buf1, [r, c], zv, mask=c < V)
        return carry

    lax.fori_loop(0, 8 * 63, zero_body, 0)

    for q in range(2):
        tok_v, w_v, tsem, wsem = ins[q]
        b = b0 + q
        pltpu.async_copy(tok_hbm.at[pl.ds(b * BTOK, BTOK)], tok_v, tsem)
        pltpu.async_copy(w_hbm.at[pl.ds(b * BTOK, BTOK)], w_v, wsem)

    def pair_body(j, carry):
        for q in range(2):
            tok_v, w_v, tsem, wsem = ins[q]
            b = b0 + j * 2 + q
            pltpu.make_async_copy(
                tok_hbm.at[pl.ds(b * BTOK, BTOK)], tok_v, tsem).wait()
            pltpu.make_async_copy(
                w_hbm.at[pl.ds(b * BTOK, BTOK)], w_v, wsem).wait()

            for lt in range(4):
                p = lt % 2
                buf, tsv, osem = bufs[p], tsvs[p], osems[p]
                prev_lt = lt - 2 if lt >= 2 else lt + 2
                prev_b = b if lt >= 2 else b - 1

                def _reclaim(prev_b=prev_b, prev_lt=prev_lt, buf=buf,
                             tsv=tsv, osem=osem):
                    pltpu.make_async_copy(
                        _buf_slice(buf, prev_lt),
                        _out_slice(out_hbm, prev_b, prev_lt), osem).wait()
                    _rezero(buf, tsv, UNIT_ROWS[prev_lt])

                if lt >= 2 or q == 1:
                    _reclaim()
                else:
                    pl.when(j > 0)(_reclaim)

                _process_unit(tok_v, w_v, buf, tsv, lt)
                if lt == 3:
                    @pl.when(j < JP - 1)
                    def _prefetch():
                        nb = b + 2
                        pltpu.async_copy(
                            tok_hbm.at[pl.ds(nb * BTOK, BTOK)], tok_v, tsem)
                        pltpu.async_copy(
                            w_hbm.at[pl.ds(nb * BTOK, BTOK)], w_v, wsem)
                pltpu.async_copy(_buf_slice(buf, lt),
                                 _out_slice(out_hbm, b, lt), osem)
        return carry

    lax.fori_loop(0, JP, pair_body, 0)

    blast = b0 + BPW - 1
    for lt in (2, 3):
        p = lt % 2
        pltpu.make_async_copy(_buf_slice(bufs[p], lt),
                              _out_slice(out_hbm, blast, lt), osems[p]).wait()


@jax.jit
def kernel(token_ids, weights):
    tok2 = token_ids.reshape(ROWS, T)
    w2 = weights.reshape(ROWS, T)
    tok_pad = jnp.concatenate(
        [tok2, jnp.broadcast_to(tok2[:, :1], (ROWS, TP - T))], axis=1)
    w_pad = jnp.concatenate(
        [w2, jnp.zeros((ROWS, TP - T), jnp.float32)], axis=1)
    tok_g = tok_pad.reshape(ROWS * TP)
    w_g = w_pad.reshape(ROWS * TP)

    mesh = plsc.VectorSubcoreMesh(
        core_axis_name="c", subcore_axis_name="s", num_cores=NC,
        num_subcores=NS)
    return pl.kernel(
        _sc_body,
        out_type=jax.ShapeDtypeStruct((B, L, V), jnp.float32),
        mesh=mesh,
        compiler_params=pltpu.CompilerParams(needs_layout_passes=False),
        scratch_types=[
            pltpu.VMEM((BTOK,), jnp.int32),
            pltpu.VMEM((BTOK,), jnp.int32),
            pltpu.VMEM((BTOK,), jnp.float32),
            pltpu.VMEM((BTOK,), jnp.float32),
            pltpu.VMEM((8, V), jnp.float32),
            pltpu.VMEM((8, V), jnp.float32),
            pltpu.VMEM((8 * TP,), jnp.int32),
            pltpu.VMEM((8 * TP,), jnp.int32),
            pltpu.SemaphoreType.DMA,
            pltpu.SemaphoreType.DMA,
            pltpu.SemaphoreType.DMA,
            pltpu.SemaphoreType.DMA,
            pltpu.SemaphoreType.DMA,
            pltpu.SemaphoreType.DMA,
        ],
    )(tok_g, w_g)


# speedup vs baseline: 12.4842x; 1.3860x over previous
import jax
import jax.numpy as jnp
from jax import lax
from jax.experimental import pallas as pl
from jax.experimental.pallas import tpu as pltpu
from jax.experimental.pallas import tpu_sc as plsc

B, L, T, V = 1024, 26, 50, 1001
LT_FULL = L // 8
LTAIL = L - 8 * LT_FULL
NC, NS = 2, 16
WORKERS = NC * NS
BPW = B // WORKERS
JP = BPW // 2
VPR = (T + 15) // 16
MAGIC = 0x5F3759DF

UNIT_ROWS = (8, 8, 8, LTAIL)
UNIT_L0 = (0, 8, 16, 24)


def _row_vecs(tok_v, w_v, l, lanes):
    lv = jnp.full((16,), l, jnp.int32)
    toks, ws = [], []
    for m in range(VPR):
        cols = jnp.minimum(m * 16 + lanes, T - 1)
        tok = plsc.load_gather(tok_v, [lv, cols])
        w = plsc.load_gather(w_v, [lv, cols])
        if m == VPR - 1:
            w = jnp.where(m * 16 + lanes < T, w, 0.0)
        toks.append(tok)
        ws.append(w)
    return toks, ws


def _process_unit(tok_v, w_v, buf, tsave, lt, lanes):
    nrows = UNIT_ROWS[lt]
    rows = []
    for r in range(nrows):
        toks, ws = _row_vecs(tok_v, w_v, UNIT_L0[lt] + r, lanes)
        rv = jnp.full((16,), r, jnp.int32)
        rows.append((rv, toks, ws))
        for m in range(VPR):
            tsave[pl.ds((r * VPR + m) * 16, 16)] = toks[m]
            plsc.addupdate_scatter(buf, [rv, toks[m]], ws[m])
    for r in range(nrows):
        rv, toks, ws = rows[r]
        cs = [plsc.load_gather(buf, [rv, toks[m]]) for m in range(VPR)]
        part = jnp.zeros((16,), jnp.float32)
        for m in range(VPR):
            part = part + cs[m] * ws[m]
        s = jnp.maximum(jnp.sum(part), 1e-12)
        xv = jnp.broadcast_to(s, (16,))
        iv = jnp.int32(MAGIC) - lax.shift_right_logical(
            plsc.bitcast(xv, jnp.int32), 1)
        y = plsc.bitcast(iv, jnp.float32)
        for _ in range(3):
            y = y * (1.5 - 0.5 * xv * y * y)
        for m in range(VPR):
            plsc.store_scatter(buf, [rv, toks[m]], cs[m] * y)


def _rezero(buf, tsave, nrows):
    zv = jnp.zeros((16,), jnp.float32)
    for r in range(nrows):
        rv = jnp.full((16,), r, jnp.int32)
        for m in range(VPR):
            tok = tsave[pl.ds((r * VPR + m) * 16, 16)]
            plsc.store_scatter(buf, [rv, tok], zv)


def _out_slice(out_hbm, b, lt):
    return out_hbm.at[b, pl.ds(UNIT_L0[lt], UNIT_ROWS[lt]), :]


def _buf_slice(buf, lt):
    return buf.at[pl.ds(0, UNIT_ROWS[lt])] if UNIT_ROWS[lt] != 8 else buf


def _sc_body(tok_hbm, w_hbm, out_hbm,
             tok0, tok1, w0, w1, buf0, buf1, tsv0, tsv1,
             osem0, osem1, tsem0, tsem1, wsem0, wsem1):
    wid = lax.axis_index("s") * NC + lax.axis_index("c")
    b0 = wid * BPW
    bufs = (buf0, buf1)
    tsvs = (tsv0, tsv1)
    osems = (osem0, osem1)
    ins = ((tok0, w0, tsem0, wsem0), (tok1, w1, tsem1, wsem1))

    lanes = lax.iota(jnp.int32, 16)

    def zero_body(i, carry):
        r = jnp.broadcast_to(i // 63, (16,))
        c = (i % 63) * 16 + lanes
        zv = jnp.zeros((16,), jnp.float32)
        plsc.store_scatter(buf0, [r, c], zv, mask=c < V)
        plsc.store_scatter(buf1, [r, c], zv, mask=c < V)
        return carry

    lax.fori_loop(0, 8 * 63, zero_body, 0)

    for q in range(2):
        tok_v, w_v, tsem, wsem = ins[q]
        b = b0 + q
        pltpu.async_copy(tok_hbm.at[b], tok_v, tsem)
        pltpu.async_copy(w_hbm.at[b], w_v, wsem)

    def pair_body(j, carry):
        for q in range(2):
            tok_v, w_v, tsem, wsem = ins[q]
            b = b0 + j * 2 + q
            pltpu.make_async_copy(tok_hbm.at[b], tok_v, tsem).wait()
            pltpu.make_async_copy(w_hbm.at[b], w_v, wsem).wait()

            for lt in range(4):
                p = lt % 2
                buf, tsv, osem = bufs[p], tsvs[p], osems[p]
                prev_lt = lt - 2 if lt >= 2 else lt + 2
                prev_b = b if lt >= 2 else b - 1

                def _reclaim(prev_b=prev_b, prev_lt=prev_lt, buf=buf,
                             tsv=tsv, osem=osem):
                    pltpu.make_async_copy(
                        _buf_slice(buf, prev_lt),
                        _out_slice(out_hbm, prev_b, prev_lt), osem).wait()
                    _rezero(buf, tsv, UNIT_ROWS[prev_lt])

                if lt >= 2 or q == 1:
                    _reclaim()
                else:
                    pl.when(j > 0)(_reclaim)

                _process_unit(tok_v, w_v, buf, tsv, lt, lanes)
                if lt == 3:
                    @pl.when(j < JP - 1)
                    def _prefetch():
                        pltpu.async_copy(tok_hbm.at[b + 2], tok_v, tsem)
                        pltpu.async_copy(w_hbm.at[b + 2], w_v, wsem)
                pltpu.async_copy(_buf_slice(buf, lt),
                                 _out_slice(out_hbm, b, lt), osem)
        return carry

    lax.fori_loop(0, JP, pair_body, 0)

    blast = b0 + BPW - 1
    for lt in (2, 3):
        p = lt % 2
        pltpu.make_async_copy(_buf_slice(bufs[p], lt),
                              _out_slice(out_hbm, blast, lt), osems[p]).wait()


@jax.jit
def kernel(token_ids, weights):
    mesh = plsc.VectorSubcoreMesh(
        core_axis_name="c", subcore_axis_name="s", num_cores=NC,
        num_subcores=NS)
    return pl.kernel(
        _sc_body,
        out_type=jax.ShapeDtypeStruct((B, L, V), jnp.float32),
        mesh=mesh,
        compiler_params=pltpu.CompilerParams(needs_layout_passes=False),
        scratch_types=[
            pltpu.VMEM((L, T), jnp.int32),
            pltpu.VMEM((L, T), jnp.int32),
            pltpu.VMEM((L, T), jnp.float32),
            pltpu.VMEM((L, T), jnp.float32),
            pltpu.VMEM((8, V), jnp.float32),
            pltpu.VMEM((8, V), jnp.float32),
            pltpu.VMEM((8 * VPR * 16,), jnp.int32),
            pltpu.VMEM((8 * VPR * 16,), jnp.int32),
            pltpu.SemaphoreType.DMA,
            pltpu.SemaphoreType.DMA,
            pltpu.SemaphoreType.DMA,
            pltpu.SemaphoreType.DMA,
            pltpu.SemaphoreType.DMA,
            pltpu.SemaphoreType.DMA,
        ],
    )(token_ids, weights)
